# baseline (speedup 1.0000x reference)
import functools
import jax
import jax.numpy as jnp
from jax.experimental import pallas as pl
from jax.experimental.pallas import tpu as pltpu

TOKENS = 8192
INPUT_DIM = 1024
HIDDEN_DIM = 256
POOL_SIZE = 16384
K = 1024
MIN_P = 100.0
MAX_P = 1024.0

ROW_BLK = 256
COL_BLK = 4096


def _hs_complexity_body(x_ref, W1_ref, b1_ref, W2_ref, b2_ref, S1_ref, bs1_ref,
                        hs_ref, comp_ref):
    x = x_ref[...]
    h = jnp.maximum(jnp.dot(x, W1_ref[...], preferred_element_type=jnp.float32)
                    + b1_ref[...], 0.0)
    logit = jnp.dot(h, W2_ref[...], preferred_element_type=jnp.float32) + b2_ref[...]
    comp_ref[...] = jax.nn.sigmoid(logit)
    hs_ref[...] = jnp.maximum(
        jnp.dot(x, S1_ref[...], preferred_element_type=jnp.float32) + bs1_ref[...], 0.0)


def _scores_body(hs_ref, S2_ref, bs2_ref, out_ref):
    out_ref[...] = (jnp.dot(hs_ref[...], S2_ref[...],
                            preferred_element_type=jnp.float32) + bs2_ref[...])


def kernel(x, W1, b1, W2, b2, S1, bs1, S2, bs2):
    n_rb = TOKENS // ROW_BLK
    hs, comp = pl.pallas_call(
        _hs_complexity_body,
        grid=(n_rb,),
        in_specs=[
            pl.BlockSpec((ROW_BLK, INPUT_DIM), lambda i: (i, 0)),
            pl.BlockSpec((INPUT_DIM, 128), lambda i: (0, 0)),
            pl.BlockSpec((128,), lambda i: (0,)),
            pl.BlockSpec((128, 1), lambda i: (0, 0)),
            pl.BlockSpec((1,), lambda i: (0,)),
            pl.BlockSpec((INPUT_DIM, HIDDEN_DIM), lambda i: (0, 0)),
            pl.BlockSpec((HIDDEN_DIM,), lambda i: (0,)),
        ],
        out_specs=[
            pl.BlockSpec((ROW_BLK, HIDDEN_DIM), lambda i: (i, 0)),
            pl.BlockSpec((ROW_BLK, 1), lambda i: (i, 0)),
        ],
        out_shape=[
            jax.ShapeDtypeStruct((TOKENS, HIDDEN_DIM), jnp.float32),
            jax.ShapeDtypeStruct((TOKENS, 1), jnp.float32),
        ],
    )(x, W1, b1, W2, b2, S1, bs1)

    n_cb = POOL_SIZE // COL_BLK
    scores = pl.pallas_call(
        _scores_body,
        grid=(n_rb, n_cb),
        in_specs=[
            pl.BlockSpec((ROW_BLK, HIDDEN_DIM), lambda i, j: (i, 0)),
            pl.BlockSpec((HIDDEN_DIM, COL_BLK), lambda i, j: (0, j)),
            pl.BlockSpec((COL_BLK,), lambda i, j: (j,)),
        ],
        out_specs=pl.BlockSpec((ROW_BLK, COL_BLK), lambda i, j: (i, j)),
        out_shape=jax.ShapeDtypeStruct((TOKENS, POOL_SIZE), jnp.float32),
    )(hs, S2, bs2)

    top_scores, indices = jax.lax.top_k(scores, K)
    weights = jax.nn.softmax(top_scores, axis=-1)
    scale = jnp.power(comp, 2.0)
    budgets = jnp.round(jnp.clip(MIN_P + (MAX_P - MIN_P) * scale, MIN_P, MAX_P)).astype(jnp.int32)
    mask = (jnp.arange(K) < budgets).astype(jnp.float32)
    weights = weights * mask
    return indices, weights, mask, comp
